# Initial kernel scaffold; baseline (speedup 1.0000x reference)
import jax
import jax.numpy as jnp
from jax.experimental import pallas as pl


def kernel(x, edge_index, W1, a_src1, a_dst1, b1, gamma1, beta1, W2, a_src2, a_dst2, b2, gamma2, beta2, W3, a_src3, a_dst3, b3):
    raise NotImplementedError("write your pallas kernel here")


# speedup vs baseline: 16.7443x; 16.7443x over previous
import functools

import jax
import jax.numpy as jnp
from jax import lax
from jax.experimental import pallas as pl
from jax.experimental.pallas import tpu as pltpu
from jax.experimental.pallas import tpu_sc as plsc

N = 10000
D = 256
H = 8
C = 32
NP = 10240
BM = 512
GRID_M = NP // BM
NS = 16
NPA = 10016
ROWS_PER_TILE = NPA // NS
K = 128
CH = 84
ET = NS * CH * K


def _tc_entry(xp, W, As, Ad):
  def body(x_ref, w_ref, as_ref, ad_ref, xl_ref, s_ref, d_ref):
    xl = jnp.dot(x_ref[...], w_ref[...], preferred_element_type=jnp.float32)
    xl_ref[0] = xl[:, :128]
    xl_ref[1] = xl[:, 128:]
    for c in range(2):
      s_ref[c] = jnp.dot(xl, as_ref[c], preferred_element_type=jnp.float32)
      d_ref[c] = jnp.dot(xl, ad_ref[c], preferred_element_type=jnp.float32)

  return pl.pallas_call(
      body,
      grid=(GRID_M,),
      in_specs=[
          pl.BlockSpec((BM, D), lambda m: (m, 0)),
          pl.BlockSpec((D, D), lambda m: (0, 0)),
          pl.BlockSpec((2, D, 16), lambda m: (0, 0, 0)),
          pl.BlockSpec((2, D, 16), lambda m: (0, 0, 0)),
      ],
      out_specs=[
          pl.BlockSpec((2, BM, 128), lambda m: (0, m, 0)),
          pl.BlockSpec((2, BM, 16), lambda m: (0, m, 0)),
          pl.BlockSpec((2, BM, 16), lambda m: (0, m, 0)),
      ],
      out_shape=[
          jax.ShapeDtypeStruct((2, NP, 128), jnp.float32),
          jax.ShapeDtypeStruct((2, NP, 16), jnp.float32),
          jax.ShapeDtypeStruct((2, NP, 16), jnp.float32),
      ],
  )(xp, W, As, Ad)


def _den_expand(den16):
  r = lax.broadcasted_iota(jnp.int32, (16, D), 0)
  cidx = lax.broadcasted_iota(jnp.int32, (16, D), 1) // C
  R = jnp.where(r == cidx, 1.0, 0.0).astype(jnp.float32)
  return jnp.dot(den16, R, preferred_element_type=jnp.float32)


def _tc_mid(agg, gp, bp, W, As, Ad):
  def body(agg_ref, g_ref, b_ref, w_ref, as_ref, ad_ref, xl_ref, s_ref, d_ref):
    num = jnp.concatenate([agg_ref[0, :, :128], agg_ref[1, :, :128]], axis=1)
    den = _den_expand(agg_ref[0, :, 128:144] + agg_ref[1, :, 128:144])
    a = num / (den + 1e-16)
    a = a * g_ref[...] + b_ref[...]
    act = jnp.where(a > 0, a, jnp.exp(a) - 1.0)
    xl = jnp.dot(act, w_ref[...], preferred_element_type=jnp.float32)
    xl_ref[0] = xl[:, :128]
    xl_ref[1] = xl[:, 128:]
    for c in range(2):
      s_ref[c] = jnp.dot(xl, as_ref[c], preferred_element_type=jnp.float32)
      d_ref[c] = jnp.dot(xl, ad_ref[c], preferred_element_type=jnp.float32)

  return pl.pallas_call(
      body,
      grid=(GRID_M,),
      in_specs=[
          pl.BlockSpec((2, BM, 144), lambda m: (0, m, 0)),
          pl.BlockSpec((1, D), lambda m: (0, 0)),
          pl.BlockSpec((1, D), lambda m: (0, 0)),
          pl.BlockSpec((D, D), lambda m: (0, 0)),
          pl.BlockSpec((2, D, 16), lambda m: (0, 0, 0)),
          pl.BlockSpec((2, D, 16), lambda m: (0, 0, 0)),
      ],
      out_specs=[
          pl.BlockSpec((2, BM, 128), lambda m: (0, m, 0)),
          pl.BlockSpec((2, BM, 16), lambda m: (0, m, 0)),
          pl.BlockSpec((2, BM, 16), lambda m: (0, m, 0)),
      ],
      out_shape=[
          jax.ShapeDtypeStruct((2, NP, 128), jnp.float32),
          jax.ShapeDtypeStruct((2, NP, 16), jnp.float32),
          jax.ShapeDtypeStruct((2, NP, 16), jnp.float32),
      ],
  )(agg, gp, bp, W, As, Ad)


def _tc_final(agg, bp):
  def body(agg_ref, b_ref, o_ref):
    num = jnp.concatenate([agg_ref[0, :, :128], agg_ref[1, :, :128]], axis=1)
    den = agg_ref[0, :, 128:129]
    o_ref[...] = num / (den + 1e-16) + b_ref[...]

  return pl.pallas_call(
      body,
      grid=(GRID_M,),
      in_specs=[
          pl.BlockSpec((2, BM, 144), lambda m: (0, m, 0)),
          pl.BlockSpec((1, D), lambda m: (0, 0)),
      ],
      out_specs=pl.BlockSpec((BM, D), lambda m: (m, 0)),
      out_shape=jax.ShapeDtypeStruct((NP, D), jnp.float32),
  )(agg, bp)


def _make_sc_edge(single_head):
  mesh = plsc.VectorSubcoreMesh(core_axis_name="c", subcore_axis_name="s")

  @functools.partial(
      pl.kernel,
      mesh=mesh,
      out_type=jax.ShapeDtypeStruct((2 * NP, 144), jnp.float32),
      scratch_types=[
          pltpu.VMEM((K,), jnp.int32),
          pltpu.VMEM((K,), jnp.int32),
          pltpu.VMEM((K,), jnp.int32),
          pltpu.VMEM((K,), jnp.int32),
          pltpu.VMEM((K, 16), jnp.float32),
          pltpu.VMEM((K, 16), jnp.float32),
          pltpu.VMEM((K, 128), jnp.float32),
          pltpu.VMEM((K, 144), jnp.float32),
          pltpu.VMEM_SHARED((NPA, 144), jnp.float32),
          pltpu.SemaphoreType.DMA,
          pltpu.SemaphoreType.DMA,
          pltpu.SemaphoreType.DMA,
      ],
      compiler_params=pltpu.CompilerParams(use_tc_tiling_on_sc=False),
  )
  def k(src_h, dst_h, as_h, ad_h, xl_h, agg_h,
        sidx, didx, sidx2, didx2, av, bv, xv, ov, acc, sem1, sem2, sem3):
    c = lax.axis_index("c")
    s = lax.axis_index("s")
    zero16 = jnp.zeros((16,), jnp.float32)

    def zrow(i, carry):
      for j in range(9):
        ov[i, pl.ds(j * 16, 16)] = zero16
      return carry
    lax.fori_loop(0, K, zrow, 0)
    r0 = s * ROWS_PER_TILE
    for t in range(ROWS_PER_TILE // K):
      pltpu.sync_copy(ov, acc.at[pl.ds(r0 + t * K, K)])
    tail = ROWS_PER_TILE % K
    if tail:
      pltpu.sync_copy(ov.at[pl.ds(0, tail)],
                      acc.at[pl.ds(r0 + (ROWS_PER_TILE // K) * K, tail)])
    plsc.subcore_barrier()

    hb = 4 * c
    lane = lax.iota(jnp.int32, 16)
    coff = c * NP
    JH = 1 if single_head else 4
    VPERH = 8 if single_head else 2

    ebase = s * (CH * K)

    def chunk(g, carry):
      b0 = ebase + g * K
      pltpu.sync_copy(src_h.at[pl.ds(b0, K)], sidx)
      pltpu.sync_copy(dst_h.at[pl.ds(b0, K)], didx)
      for j in range(K // 16):
        sidx2[pl.ds(j * 16, 16)] = sidx[pl.ds(j * 16, 16)] + coff
        didx2[pl.ds(j * 16, 16)] = didx[pl.ds(j * 16, 16)] + coff
      cp1 = pltpu.async_copy(as_h.at[sidx2], av, sem1)
      cp2 = pltpu.async_copy(ad_h.at[didx2], bv, sem2)
      cp3 = pltpu.async_copy(xl_h.at[sidx2], xv, sem3)
      cp1.wait()
      cp2.wait()
      cp3.wait()

      def edge(kk, cc):
        den = zero16
        ev = av[kk, :] + bv[kk, :]
        for jh in range(JH):
          e_v = jnp.full((16,), ev[jh], jnp.float32)
          w_v = jnp.exp(jnp.maximum(e_v, 0.2 * e_v))
          den = jnp.where(lane == hb + jh, w_v, den)
          for j in range(VPERH):
            col = (jh * VPERH + j) * 16
            ov[kk, pl.ds(col, 16)] = xv[kk, pl.ds(col, 16)] * w_v
        ov[kk, pl.ds(128, 16)] = den
        return cc
      lax.fori_loop(0, K, edge, 0)
      pltpu.sync_copy(ov, acc.at[didx], add=True)
      return carry
    lax.fori_loop(0, CH, chunk, 0)

    plsc.subcore_barrier()
    pltpu.sync_copy(acc.at[pl.ds(r0, ROWS_PER_TILE)],
                    agg_h.at[pl.ds(coff + r0, ROWS_PER_TILE)])

  return k


_sc_edge_multi = _make_sc_edge(False)
_sc_edge_single = _make_sc_edge(True)


def _alpha_mat(a, heads):
  a_flat = a.reshape(-1)
  lvec = jnp.arange(16, dtype=jnp.int32)
  if heads == 1:
    slab = a_flat[:, None] * (lvec[None, :] == 0).astype(jnp.float32)
    return jnp.stack([slab, slab])
  hvec = jnp.arange(D, dtype=jnp.int32) // C
  slabs = []
  for c in range(2):
    mask = (lvec[None, :] == hvec[:, None] - 4 * c).astype(jnp.float32)
    slabs.append(a_flat[:, None] * mask)
  return jnp.stack(slabs)


def kernel(x, edge_index, W1, a_src1, a_dst1, b1, gamma1, beta1,
           W2, a_src2, a_dst2, b2, gamma2, beta2,
           W3, a_src3, a_dst3, b3):
  f32 = jnp.float32
  xp = jnp.pad(x, ((0, NP - N), (0, 0)))
  loop = jnp.arange(N, dtype=jnp.int32)
  pad = jnp.full((ET - edge_index.shape[1] - N,), N, dtype=jnp.int32)
  src = jnp.concatenate([edge_index[0].astype(jnp.int32), loop, pad])
  dst = jnp.concatenate([edge_index[1].astype(jnp.int32), loop, pad])

  inv = 1.0 / jnp.sqrt(jnp.asarray(1.0 + 1e-5, f32))
  g1p = (gamma1 * inv)[None, :]
  b1p = (b1 * gamma1 * inv + beta1)[None, :]
  g2p = (gamma2 * inv)[None, :]
  b2p = (b2 * gamma2 * inv + beta2)[None, :]
  b3p = b3[None, :]

  As1 = _alpha_mat(a_src1, H)
  Ad1 = _alpha_mat(a_dst1, H)
  As2 = _alpha_mat(a_src2, H)
  Ad2 = _alpha_mat(a_dst2, H)
  As3 = _alpha_mat(a_src3, 1)
  Ad3 = _alpha_mat(a_dst3, 1)

  xl1, as1, ad1 = _tc_entry(xp, W1, As1, Ad1)
  agg1 = _sc_edge_multi(src, dst, as1.reshape(2 * NP, 16),
                        ad1.reshape(2 * NP, 16), xl1.reshape(2 * NP, 128))
  xl2, as2, ad2 = _tc_mid(agg1.reshape(2, NP, 144), g1p, b1p, W2, As2, Ad2)
  agg2 = _sc_edge_multi(src, dst, as2.reshape(2 * NP, 16),
                        ad2.reshape(2 * NP, 16), xl2.reshape(2 * NP, 128))
  xl3, as3, ad3 = _tc_mid(agg2.reshape(2, NP, 144), g2p, b2p, W3, As3, Ad3)
  agg3 = _sc_edge_single(src, dst, as3.reshape(2 * NP, 16),
                         ad3.reshape(2 * NP, 16), xl3.reshape(2 * NP, 128))
  out = _tc_final(agg3.reshape(2, NP, 144), b3p)
  return out[:N]


# speedup vs baseline: 19.4052x; 1.1589x over previous
import functools

import jax
import jax.numpy as jnp
from jax import lax
from jax.experimental import pallas as pl
from jax.experimental.pallas import tpu as pltpu
from jax.experimental.pallas import tpu_sc as plsc

N = 10000
D = 256
H = 8
C = 32
NP = 10240
BM = 512
GRID_M = NP // BM
NS = 16
NPA = 10016
ROWS_PER_TILE = NPA // NS
K = 64
CH = 168
ET = NS * CH * K


def _tc_entry(xp, W, As, Ad):
  def body(x_ref, w_ref, as_ref, ad_ref, xl_ref, s_ref, d_ref):
    xl = jnp.dot(x_ref[...], w_ref[...], preferred_element_type=jnp.float32)
    xl_ref[0] = xl[:, :128]
    xl_ref[1] = xl[:, 128:]
    for c in range(2):
      s_ref[c] = jnp.dot(xl, as_ref[c], preferred_element_type=jnp.float32)
      d_ref[c] = jnp.dot(xl, ad_ref[c], preferred_element_type=jnp.float32)

  return pl.pallas_call(
      body,
      grid=(GRID_M,),
      in_specs=[
          pl.BlockSpec((BM, D), lambda m: (m, 0)),
          pl.BlockSpec((D, D), lambda m: (0, 0)),
          pl.BlockSpec((2, D, 16), lambda m: (0, 0, 0)),
          pl.BlockSpec((2, D, 16), lambda m: (0, 0, 0)),
      ],
      out_specs=[
          pl.BlockSpec((2, BM, 128), lambda m: (0, m, 0)),
          pl.BlockSpec((2, BM, 16), lambda m: (0, m, 0)),
          pl.BlockSpec((2, BM, 16), lambda m: (0, m, 0)),
      ],
      out_shape=[
          jax.ShapeDtypeStruct((2, NP, 128), jnp.float32),
          jax.ShapeDtypeStruct((2, NP, 16), jnp.float32),
          jax.ShapeDtypeStruct((2, NP, 16), jnp.float32),
      ],
  )(xp, W, As, Ad)


def _den_expand(den8):
  r = lax.broadcasted_iota(jnp.int32, (8, D), 0)
  cidx = lax.broadcasted_iota(jnp.int32, (8, D), 1) // C
  R = jnp.where(r == cidx, 1.0, 0.0).astype(jnp.float32)
  return jnp.dot(den8, R, preferred_element_type=jnp.float32)


def _tc_mid(agg, gp, bp, W, As, Ad):
  def body(agg_ref, g_ref, b_ref, w_ref, as_ref, ad_ref, xl_ref, s_ref, d_ref):
    num = jnp.concatenate([agg_ref[0, :, :128], agg_ref[1, :, :128]], axis=1)
    den = _den_expand(jnp.concatenate(
        [agg_ref[0, :, 128:132], agg_ref[1, :, 128:132]], axis=1))
    a = num / (den + 1e-16)
    a = a * g_ref[...] + b_ref[...]
    act = jnp.where(a > 0, a, jnp.exp(a) - 1.0)
    xl = jnp.dot(act, w_ref[...], preferred_element_type=jnp.float32)
    xl_ref[0] = xl[:, :128]
    xl_ref[1] = xl[:, 128:]
    for c in range(2):
      s_ref[c] = jnp.dot(xl, as_ref[c], preferred_element_type=jnp.float32)
      d_ref[c] = jnp.dot(xl, ad_ref[c], preferred_element_type=jnp.float32)

  return pl.pallas_call(
      body,
      grid=(GRID_M,),
      in_specs=[
          pl.BlockSpec((2, BM, 144), lambda m: (0, m, 0)),
          pl.BlockSpec((1, D), lambda m: (0, 0)),
          pl.BlockSpec((1, D), lambda m: (0, 0)),
          pl.BlockSpec((D, D), lambda m: (0, 0)),
          pl.BlockSpec((2, D, 16), lambda m: (0, 0, 0)),
          pl.BlockSpec((2, D, 16), lambda m: (0, 0, 0)),
      ],
      out_specs=[
          pl.BlockSpec((2, BM, 128), lambda m: (0, m, 0)),
          pl.BlockSpec((2, BM, 16), lambda m: (0, m, 0)),
          pl.BlockSpec((2, BM, 16), lambda m: (0, m, 0)),
      ],
      out_shape=[
          jax.ShapeDtypeStruct((2, NP, 128), jnp.float32),
          jax.ShapeDtypeStruct((2, NP, 16), jnp.float32),
          jax.ShapeDtypeStruct((2, NP, 16), jnp.float32),
      ],
  )(agg, gp, bp, W, As, Ad)


def _tc_final(agg, bp):
  def body(agg_ref, b_ref, o_ref):
    num = jnp.concatenate([agg_ref[0, :, :128], agg_ref[1, :, :128]], axis=1)
    den = agg_ref[0, :, 128:129]
    o_ref[...] = num / (den + 1e-16) + b_ref[...]

  return pl.pallas_call(
      body,
      grid=(GRID_M,),
      in_specs=[
          pl.BlockSpec((2, BM, 144), lambda m: (0, m, 0)),
          pl.BlockSpec((1, D), lambda m: (0, 0)),
      ],
      out_specs=pl.BlockSpec((BM, D), lambda m: (m, 0)),
      out_shape=jax.ShapeDtypeStruct((NP, D), jnp.float32),
  )(agg, bp)


def _make_sc_edge(single_head):
  mesh = plsc.VectorSubcoreMesh(core_axis_name="c", subcore_axis_name="s")
  buf_types = [
      pltpu.VMEM((K,), jnp.int32),
      pltpu.VMEM((K,), jnp.int32),
      pltpu.VMEM((K,), jnp.int32),
      pltpu.VMEM((K,), jnp.int32),
      pltpu.VMEM((K, 16), jnp.float32),
      pltpu.VMEM((K, 16), jnp.float32),
      pltpu.VMEM((K, 128), jnp.float32),
      pltpu.SemaphoreType.DMA,
      pltpu.SemaphoreType.DMA,
      pltpu.SemaphoreType.DMA,
  ]

  @functools.partial(
      pl.kernel,
      mesh=mesh,
      out_type=jax.ShapeDtypeStruct((2 * NP, 144), jnp.float32),
      scratch_types=buf_types + buf_types + [
          pltpu.VMEM((K, 144), jnp.float32),
          pltpu.VMEM_SHARED((NPA, 144), jnp.float32),
      ],
      compiler_params=pltpu.CompilerParams(use_tc_tiling_on_sc=False),
  )
  def k(src_h, dst_h, as_h, ad_h, xl_h, agg_h, *refs):
    bufA = refs[0:10]
    bufB = refs[10:20]
    ov, acc = refs[20], refs[21]
    c = lax.axis_index("c")
    s = lax.axis_index("s")
    zero16 = jnp.zeros((16,), jnp.float32)

    def zrow(i, carry):
      for j in range(9):
        ov[i, pl.ds(j * 16, 16)] = zero16
      return carry
    lax.fori_loop(0, K, zrow, 0)
    r0 = s * ROWS_PER_TILE
    for t in range(ROWS_PER_TILE // K):
      pltpu.sync_copy(ov, acc.at[pl.ds(r0 + t * K, K)])
    tail = ROWS_PER_TILE % K
    if tail:
      pltpu.sync_copy(ov.at[pl.ds(0, tail)],
                      acc.at[pl.ds(r0 + (ROWS_PER_TILE // K) * K, tail)])
    plsc.subcore_barrier()

    lane = lax.iota(jnp.int32, 16)
    coff = c * NP
    JH = 1 if single_head else 4
    VPERH = 8 if single_head else 2
    lane_mask = lane < JH

    ebase = s * (CH * K)

    def prefetch(g, buf):
      si, di, si2, di2, avb, bvb, xvb, sa, sb, sx = buf
      b0 = ebase + g * K
      pltpu.sync_copy(src_h.at[pl.ds(b0, K)], si)
      pltpu.sync_copy(dst_h.at[pl.ds(b0, K)], di)
      for j in range(K // 16):
        si2[pl.ds(j * 16, 16)] = si[pl.ds(j * 16, 16)] + coff
        di2[pl.ds(j * 16, 16)] = di[pl.ds(j * 16, 16)] + coff
      pltpu.async_copy(as_h.at[si2], avb, sa)
      pltpu.async_copy(ad_h.at[di2], bvb, sb)
      pltpu.async_copy(xl_h.at[si2], xvb, sx)

    def consume(buf):
      si, di, si2, di2, avb, bvb, xvb, sa, sb, sx = buf
      pltpu.make_async_copy(as_h.at[si2], avb, sa).wait()
      pltpu.make_async_copy(ad_h.at[di2], bvb, sb).wait()
      pltpu.make_async_copy(xl_h.at[si2], xvb, sx).wait()

      def edge(kk, cc):
        ev = avb[kk, :] + bvb[kk, :]
        w16 = jnp.exp(jnp.maximum(ev, 0.2 * ev))
        ov[kk, pl.ds(128, 16)] = jnp.where(lane_mask, w16, zero16)
        for jh in range(JH):
          w_v = jnp.full((16,), w16[jh], jnp.float32)
          for j in range(VPERH):
            col = (jh * VPERH + j) * 16
            ov[kk, pl.ds(col, 16)] = xvb[kk, pl.ds(col, 16)] * w_v
        return cc
      lax.fori_loop(0, K, edge, 0)
      pltpu.sync_copy(ov, acc.at[di], add=True)

    prefetch(0, bufA)

    def pair(t, carry):
      g = 2 * t
      prefetch(g + 1, bufB)
      consume(bufA)

      @pl.when(g + 2 < CH)
      def _():
        prefetch(g + 2, bufA)
      consume(bufB)
      return carry
    lax.fori_loop(0, CH // 2, pair, 0)

    plsc.subcore_barrier()
    pltpu.sync_copy(acc.at[pl.ds(r0, ROWS_PER_TILE)],
                    agg_h.at[pl.ds(coff + r0, ROWS_PER_TILE)])

  return k


_sc_edge_multi = _make_sc_edge(False)
_sc_edge_single = _make_sc_edge(True)


def _alpha_mat(a, heads):
  a_flat = a.reshape(-1)
  lvec = jnp.arange(16, dtype=jnp.int32)
  if heads == 1:
    slab = a_flat[:, None] * (lvec[None, :] == 0).astype(jnp.float32)
    return jnp.stack([slab, slab])
  hvec = jnp.arange(D, dtype=jnp.int32) // C
  slabs = []
  for c in range(2):
    mask = (lvec[None, :] == hvec[:, None] - 4 * c).astype(jnp.float32)
    slabs.append(a_flat[:, None] * mask)
  return jnp.stack(slabs)


def kernel(x, edge_index, W1, a_src1, a_dst1, b1, gamma1, beta1,
           W2, a_src2, a_dst2, b2, gamma2, beta2,
           W3, a_src3, a_dst3, b3):
  f32 = jnp.float32
  xp = jnp.pad(x, ((0, NP - N), (0, 0)))
  loop = jnp.arange(N, dtype=jnp.int32)
  pad = jnp.full((ET - edge_index.shape[1] - N,), N, dtype=jnp.int32)
  src = jnp.concatenate([edge_index[0].astype(jnp.int32), loop, pad])
  dst = jnp.concatenate([edge_index[1].astype(jnp.int32), loop, pad])

  inv = 1.0 / jnp.sqrt(jnp.asarray(1.0 + 1e-5, f32))
  g1p = (gamma1 * inv)[None, :]
  b1p = (b1 * gamma1 * inv + beta1)[None, :]
  g2p = (gamma2 * inv)[None, :]
  b2p = (b2 * gamma2 * inv + beta2)[None, :]
  b3p = b3[None, :]

  As1 = _alpha_mat(a_src1, H)
  Ad1 = _alpha_mat(a_dst1, H)
  As2 = _alpha_mat(a_src2, H)
  Ad2 = _alpha_mat(a_dst2, H)
  As3 = _alpha_mat(a_src3, 1)
  Ad3 = _alpha_mat(a_dst3, 1)

  xl1, as1, ad1 = _tc_entry(xp, W1, As1, Ad1)
  agg1 = _sc_edge_multi(src, dst, as1.reshape(2 * NP, 16),
                        ad1.reshape(2 * NP, 16), xl1.reshape(2 * NP, 128))
  xl2, as2, ad2 = _tc_mid(agg1.reshape(2, NP, 144), g1p, b1p, W2, As2, Ad2)
  agg2 = _sc_edge_multi(src, dst, as2.reshape(2 * NP, 16),
                        ad2.reshape(2 * NP, 16), xl2.reshape(2 * NP, 128))
  xl3, as3, ad3 = _tc_mid(agg2.reshape(2, NP, 144), g2p, b2p, W3, As3, Ad3)
  agg3 = _sc_edge_single(src, dst, as3.reshape(2 * NP, 16),
                         ad3.reshape(2 * NP, 16), xl3.reshape(2 * NP, 128))
  out = _tc_final(agg3.reshape(2, NP, 144), b3p)
  return out[:N]


# speedup vs baseline: 20.9366x; 1.0789x over previous
import functools

import jax
import jax.numpy as jnp
from jax import lax
from jax.experimental import pallas as pl
from jax.experimental.pallas import tpu as pltpu
from jax.experimental.pallas import tpu_sc as plsc

N = 10000
D = 256
H = 8
C = 32
NP = 10240
BM = 512
GRID_M = NP // BM
NS = 16
NPA = 10016
ROWS_PER_TILE = NPA // NS
K = 64
CH = 168
ET = NS * CH * K


def _tc_entry(xp, W, As, Ad):
  def body(x_ref, w_ref, as_ref, ad_ref, xl_ref, s_ref, d_ref):
    xl = jnp.dot(x_ref[...], w_ref[...], preferred_element_type=jnp.float32)
    xl_ref[0] = xl[:, :128]
    xl_ref[1] = xl[:, 128:]
    for c in range(2):
      s_ref[c] = jnp.dot(xl, as_ref[c], preferred_element_type=jnp.float32)
      d_ref[c] = jnp.dot(xl, ad_ref[c], preferred_element_type=jnp.float32)

  return pl.pallas_call(
      body,
      grid=(GRID_M,),
      in_specs=[
          pl.BlockSpec((BM, D), lambda m: (m, 0)),
          pl.BlockSpec((D, D), lambda m: (0, 0)),
          pl.BlockSpec((2, D, 16), lambda m: (0, 0, 0)),
          pl.BlockSpec((2, D, 16), lambda m: (0, 0, 0)),
      ],
      out_specs=[
          pl.BlockSpec((2, BM, 128), lambda m: (0, m, 0)),
          pl.BlockSpec((2, BM, 16), lambda m: (0, m, 0)),
          pl.BlockSpec((2, BM, 16), lambda m: (0, m, 0)),
      ],
      out_shape=[
          jax.ShapeDtypeStruct((2, NP, 128), jnp.float32),
          jax.ShapeDtypeStruct((2, NP, 16), jnp.float32),
          jax.ShapeDtypeStruct((2, NP, 16), jnp.float32),
      ],
  )(xp, W, As, Ad)


def _den_expand(den8):
  r = lax.broadcasted_iota(jnp.int32, (8, D), 0)
  cidx = lax.broadcasted_iota(jnp.int32, (8, D), 1) // C
  R = jnp.where(r == cidx, 1.0, 0.0).astype(jnp.float32)
  return jnp.dot(den8, R, preferred_element_type=jnp.float32)


def _tc_mid(agg, gp, bp, W, As, Ad):
  def body(agg_ref, g_ref, b_ref, w_ref, as_ref, ad_ref, xl_ref, s_ref, d_ref):
    num = jnp.concatenate([agg_ref[0, :, :128], agg_ref[1, :, :128]], axis=1)
    den = _den_expand(jnp.concatenate(
        [agg_ref[0, :, 128:132], agg_ref[1, :, 128:132]], axis=1))
    a = num / (den + 1e-16)
    a = a * g_ref[...] + b_ref[...]
    act = jnp.where(a > 0, a, jnp.exp(a) - 1.0)
    xl = jnp.dot(act, w_ref[...], preferred_element_type=jnp.float32)
    xl_ref[0] = xl[:, :128]
    xl_ref[1] = xl[:, 128:]
    for c in range(2):
      s_ref[c] = jnp.dot(xl, as_ref[c], preferred_element_type=jnp.float32)
      d_ref[c] = jnp.dot(xl, ad_ref[c], preferred_element_type=jnp.float32)

  return pl.pallas_call(
      body,
      grid=(GRID_M,),
      in_specs=[
          pl.BlockSpec((2, BM, 144), lambda m: (0, m, 0)),
          pl.BlockSpec((1, D), lambda m: (0, 0)),
          pl.BlockSpec((1, D), lambda m: (0, 0)),
          pl.BlockSpec((D, D), lambda m: (0, 0)),
          pl.BlockSpec((2, D, 16), lambda m: (0, 0, 0)),
          pl.BlockSpec((2, D, 16), lambda m: (0, 0, 0)),
      ],
      out_specs=[
          pl.BlockSpec((2, BM, 128), lambda m: (0, m, 0)),
          pl.BlockSpec((2, BM, 16), lambda m: (0, m, 0)),
          pl.BlockSpec((2, BM, 16), lambda m: (0, m, 0)),
      ],
      out_shape=[
          jax.ShapeDtypeStruct((2, NP, 128), jnp.float32),
          jax.ShapeDtypeStruct((2, NP, 16), jnp.float32),
          jax.ShapeDtypeStruct((2, NP, 16), jnp.float32),
      ],
  )(agg, gp, bp, W, As, Ad)


def _tc_final(agg, bp):
  def body(agg_ref, b_ref, o_ref):
    num = jnp.concatenate([agg_ref[0, :, :128], agg_ref[1, :, :128]], axis=1)
    den = agg_ref[0, :, 128:129]
    o_ref[...] = num / (den + 1e-16) + b_ref[...]

  return pl.pallas_call(
      body,
      grid=(GRID_M,),
      in_specs=[
          pl.BlockSpec((2, BM, 144), lambda m: (0, m, 0)),
          pl.BlockSpec((1, D), lambda m: (0, 0)),
      ],
      out_specs=pl.BlockSpec((BM, D), lambda m: (m, 0)),
      out_shape=jax.ShapeDtypeStruct((NP, D), jnp.float32),
  )(agg, bp)


def _make_sc_edge(single_head):
  mesh = plsc.VectorSubcoreMesh(core_axis_name="c", subcore_axis_name="s")
  buf_types = [
      pltpu.VMEM((K,), jnp.int32),
      pltpu.VMEM((K,), jnp.int32),
      pltpu.VMEM((K,), jnp.int32),
      pltpu.VMEM((K,), jnp.int32),
      pltpu.VMEM((K, 16), jnp.float32),
      pltpu.VMEM((K, 16), jnp.float32),
      pltpu.VMEM((K, 128), jnp.float32),
      pltpu.VMEM((K,), jnp.int32),
      pltpu.VMEM((K, 144), jnp.float32),
      pltpu.SemaphoreType.DMA,
      pltpu.SemaphoreType.DMA,
      pltpu.SemaphoreType.DMA,
      pltpu.SemaphoreType.DMA,
  ]

  @functools.partial(
      pl.kernel,
      mesh=mesh,
      out_type=jax.ShapeDtypeStruct((2 * NP, 144), jnp.float32),
      scratch_types=buf_types + buf_types + [
          pltpu.VMEM_SHARED((NPA, 144), jnp.float32),
      ],
      compiler_params=pltpu.CompilerParams(use_tc_tiling_on_sc=False),
  )
  def k(src_h, dst_h, as_h, ad_h, xl_h, agg_h, *refs):
    nb = len(refs) // 2
    bufA = refs[0:nb]
    bufB = refs[nb:2 * nb]
    ov = bufA[8]
    acc = refs[2 * nb]
    c = lax.axis_index("c")
    s = lax.axis_index("s")
    zero16 = jnp.zeros((16,), jnp.float32)

    def zrow(i, carry):
      for j in range(9):
        ov[i, pl.ds(j * 16, 16)] = zero16
      return carry
    lax.fori_loop(0, K, zrow, 0)
    r0 = s * ROWS_PER_TILE
    for t in range(ROWS_PER_TILE // K):
      pltpu.sync_copy(ov, acc.at[pl.ds(r0 + t * K, K)])
    tail = ROWS_PER_TILE % K
    if tail:
      pltpu.sync_copy(ov.at[pl.ds(0, tail)],
                      acc.at[pl.ds(r0 + (ROWS_PER_TILE // K) * K, tail)])
    plsc.subcore_barrier()

    lane = lax.iota(jnp.int32, 16)
    coff = c * NP
    JH = 1 if single_head else 4
    VPERH = 8 if single_head else 2
    lane_mask = lane < JH

    ebase = s * (CH * K)

    def prefetch(g, buf):
      si, di, si2, di2, avb, bvb, xvb, dis, ovb, sa, sb, sx, ss = buf
      b0 = ebase + g * K
      pltpu.sync_copy(src_h.at[pl.ds(b0, K)], si)
      pltpu.sync_copy(dst_h.at[pl.ds(b0, K)], di)
      for j in range(K // 16):
        si2[pl.ds(j * 16, 16)] = si[pl.ds(j * 16, 16)] + coff
        di2[pl.ds(j * 16, 16)] = di[pl.ds(j * 16, 16)] + coff
      pltpu.async_copy(as_h.at[si2], avb, sa)
      pltpu.async_copy(ad_h.at[di2], bvb, sb)
      pltpu.async_copy(xl_h.at[si2], xvb, sx)

    def consume(g, t, buf):
      si, di, si2, di2, avb, bvb, xvb, dis, ovb, sa, sb, sx, ss = buf
      pltpu.make_async_copy(as_h.at[si2], avb, sa).wait()
      pltpu.make_async_copy(ad_h.at[di2], bvb, sb).wait()
      pltpu.make_async_copy(xl_h.at[si2], xvb, sx).wait()

      @pl.when(t > 0)
      def _():
        pltpu.make_async_copy(ovb, acc.at[dis], ss).wait()

      def edge(kk, cc):
        ev = avb[kk, :] + bvb[kk, :]
        w16 = jnp.exp(jnp.maximum(ev, 0.2 * ev))
        ovb[kk, pl.ds(128, 16)] = jnp.where(lane_mask, w16, zero16)
        for jh in range(JH):
          w_v = jnp.full((16,), w16[jh], jnp.float32)
          for j in range(VPERH):
            col = (jh * VPERH + j) * 16
            ovb[kk, pl.ds(col, 16)] = xvb[kk, pl.ds(col, 16)] * w_v
        return cc
      lax.fori_loop(0, K, edge, 0)
      for j in range(K // 16):
        dis[pl.ds(j * 16, 16)] = di[pl.ds(j * 16, 16)]
      pltpu.async_copy(ovb, acc.at[dis], ss, add=True)

    prefetch(0, bufA)
    prefetch(1, bufB)

    def pair(t, carry):
      g = 2 * t
      consume(g, t, bufA)

      @pl.when(g + 2 < CH)
      def _():
        prefetch(g + 2, bufA)
      consume(g + 1, t, bufB)

      @pl.when(g + 3 < CH)
      def _():
        prefetch(g + 3, bufB)
      return carry
    lax.fori_loop(0, CH // 2, pair, 0)

    siA, diA, si2A, di2A, avA, bvA, xvA, disA, ovA, saA, sbA, sxA, ssA = bufA
    siB, diB, si2B, di2B, avB, bvB, xvB, disB, ovB, saB, sbB, sxB, ssB = bufB
    pltpu.make_async_copy(ovA, acc.at[disA], ssA).wait()
    pltpu.make_async_copy(ovB, acc.at[disB], ssB).wait()

    plsc.subcore_barrier()
    pltpu.sync_copy(acc.at[pl.ds(r0, ROWS_PER_TILE)],
                    agg_h.at[pl.ds(coff + r0, ROWS_PER_TILE)])

  return k


_sc_edge_multi = _make_sc_edge(False)
_sc_edge_single = _make_sc_edge(True)


def _alpha_mat(a, heads):
  a_flat = a.reshape(-1)
  lvec = jnp.arange(16, dtype=jnp.int32)
  if heads == 1:
    slab = a_flat[:, None] * (lvec[None, :] == 0).astype(jnp.float32)
    return jnp.stack([slab, slab])
  hvec = jnp.arange(D, dtype=jnp.int32) // C
  slabs = []
  for c in range(2):
    mask = (lvec[None, :] == hvec[:, None] - 4 * c).astype(jnp.float32)
    slabs.append(a_flat[:, None] * mask)
  return jnp.stack(slabs)


def kernel(x, edge_index, W1, a_src1, a_dst1, b1, gamma1, beta1,
           W2, a_src2, a_dst2, b2, gamma2, beta2,
           W3, a_src3, a_dst3, b3):
  f32 = jnp.float32
  xp = jnp.pad(x, ((0, NP - N), (0, 0)))
  loop = jnp.arange(N, dtype=jnp.int32)
  pad = jnp.full((ET - edge_index.shape[1] - N,), N, dtype=jnp.int32)
  src = jnp.concatenate([edge_index[0].astype(jnp.int32), loop, pad])
  dst = jnp.concatenate([edge_index[1].astype(jnp.int32), loop, pad])

  inv = 1.0 / jnp.sqrt(jnp.asarray(1.0 + 1e-5, f32))
  g1p = (gamma1 * inv)[None, :]
  b1p = (b1 * gamma1 * inv + beta1)[None, :]
  g2p = (gamma2 * inv)[None, :]
  b2p = (b2 * gamma2 * inv + beta2)[None, :]
  b3p = b3[None, :]

  As1 = _alpha_mat(a_src1, H)
  Ad1 = _alpha_mat(a_dst1, H)
  As2 = _alpha_mat(a_src2, H)
  Ad2 = _alpha_mat(a_dst2, H)
  As3 = _alpha_mat(a_src3, 1)
  Ad3 = _alpha_mat(a_dst3, 1)

  xl1, as1, ad1 = _tc_entry(xp, W1, As1, Ad1)
  agg1 = _sc_edge_multi(src, dst, as1.reshape(2 * NP, 16),
                        ad1.reshape(2 * NP, 16), xl1.reshape(2 * NP, 128))
  xl2, as2, ad2 = _tc_mid(agg1.reshape(2, NP, 144), g1p, b1p, W2, As2, Ad2)
  agg2 = _sc_edge_multi(src, dst, as2.reshape(2 * NP, 16),
                        ad2.reshape(2 * NP, 16), xl2.reshape(2 * NP, 128))
  xl3, as3, ad3 = _tc_mid(agg2.reshape(2, NP, 144), g2p, b2p, W3, As3, Ad3)
  agg3 = _sc_edge_single(src, dst, as3.reshape(2 * NP, 16),
                         ad3.reshape(2 * NP, 16), xl3.reshape(2 * NP, 128))
  out = _tc_final(agg3.reshape(2, NP, 144), b3p)
  return out[:N]


# speedup vs baseline: 42.4126x; 2.0258x over previous
import functools

import jax
import jax.numpy as jnp
from jax import lax
from jax.experimental import pallas as pl
from jax.experimental.pallas import tpu as pltpu
from jax.experimental.pallas import tpu_sc as plsc

N = 10000
D = 256
H = 8
C = 32
NP = 10240
BM = 512
GRID_M = NP // BM
NS = 16
NPA = 10016
ROWS_PER_TILE = NPA // NS
K = 64
CH = 168
ET = NS * CH * K


def _tc_entry(xp, W, As, Ad):
  def body(x_ref, w_ref, as_ref, ad_ref, xl_ref, s_ref, d_ref):
    xl = jnp.dot(x_ref[...], w_ref[...], preferred_element_type=jnp.float32)
    xl_ref[0] = xl[:, :128]
    xl_ref[1] = xl[:, 128:]
    for c in range(2):
      s_ref[c] = jnp.dot(xl, as_ref[c], preferred_element_type=jnp.float32)
      d_ref[c] = jnp.dot(xl, ad_ref[c], preferred_element_type=jnp.float32)

  return pl.pallas_call(
      body,
      grid=(GRID_M,),
      in_specs=[
          pl.BlockSpec((BM, D), lambda m: (m, 0)),
          pl.BlockSpec((D, D), lambda m: (0, 0)),
          pl.BlockSpec((2, D, 16), lambda m: (0, 0, 0)),
          pl.BlockSpec((2, D, 16), lambda m: (0, 0, 0)),
      ],
      out_specs=[
          pl.BlockSpec((2, BM, 128), lambda m: (0, m, 0)),
          pl.BlockSpec((2, BM, 16), lambda m: (0, m, 0)),
          pl.BlockSpec((2, BM, 16), lambda m: (0, m, 0)),
      ],
      out_shape=[
          jax.ShapeDtypeStruct((2, NP, 128), jnp.float32),
          jax.ShapeDtypeStruct((2, NP, 16), jnp.float32),
          jax.ShapeDtypeStruct((2, NP, 16), jnp.float32),
      ],
  )(xp, W, As, Ad)


def _den_expand(den8):
  r = lax.broadcasted_iota(jnp.int32, (8, D), 0)
  cidx = lax.broadcasted_iota(jnp.int32, (8, D), 1) // C
  R = jnp.where(r == cidx, 1.0, 0.0).astype(jnp.float32)
  return jnp.dot(den8, R, preferred_element_type=jnp.float32)


def _tc_mid(agg, gp, bp, W, As, Ad):
  def body(agg_ref, g_ref, b_ref, w_ref, as_ref, ad_ref, xl_ref, s_ref, d_ref):
    num = jnp.concatenate([agg_ref[0, :, :128], agg_ref[1, :, :128]], axis=1)
    den = _den_expand(jnp.concatenate(
        [agg_ref[0, :, 128:132], agg_ref[1, :, 128:132]], axis=1))
    a = num / (den + 1e-16)
    a = a * g_ref[...] + b_ref[...]
    act = jnp.where(a > 0, a, jnp.exp(a) - 1.0)
    xl = jnp.dot(act, w_ref[...], preferred_element_type=jnp.float32)
    xl_ref[0] = xl[:, :128]
    xl_ref[1] = xl[:, 128:]
    for c in range(2):
      s_ref[c] = jnp.dot(xl, as_ref[c], preferred_element_type=jnp.float32)
      d_ref[c] = jnp.dot(xl, ad_ref[c], preferred_element_type=jnp.float32)

  return pl.pallas_call(
      body,
      grid=(GRID_M,),
      in_specs=[
          pl.BlockSpec((2, BM, 144), lambda m: (0, m, 0)),
          pl.BlockSpec((1, D), lambda m: (0, 0)),
          pl.BlockSpec((1, D), lambda m: (0, 0)),
          pl.BlockSpec((D, D), lambda m: (0, 0)),
          pl.BlockSpec((2, D, 16), lambda m: (0, 0, 0)),
          pl.BlockSpec((2, D, 16), lambda m: (0, 0, 0)),
      ],
      out_specs=[
          pl.BlockSpec((2, BM, 128), lambda m: (0, m, 0)),
          pl.BlockSpec((2, BM, 16), lambda m: (0, m, 0)),
          pl.BlockSpec((2, BM, 16), lambda m: (0, m, 0)),
      ],
      out_shape=[
          jax.ShapeDtypeStruct((2, NP, 128), jnp.float32),
          jax.ShapeDtypeStruct((2, NP, 16), jnp.float32),
          jax.ShapeDtypeStruct((2, NP, 16), jnp.float32),
      ],
  )(agg, gp, bp, W, As, Ad)


def _tc_final(agg, bp):
  def body(agg_ref, b_ref, o_ref):
    num = jnp.concatenate([agg_ref[0, :, :128], agg_ref[1, :, :128]], axis=1)
    den = agg_ref[0, :, 128:129]
    o_ref[...] = num / (den + 1e-16) + b_ref[...]

  return pl.pallas_call(
      body,
      grid=(GRID_M,),
      in_specs=[
          pl.BlockSpec((2, BM, 144), lambda m: (0, m, 0)),
          pl.BlockSpec((1, D), lambda m: (0, 0)),
      ],
      out_specs=pl.BlockSpec((BM, D), lambda m: (m, 0)),
      out_shape=jax.ShapeDtypeStruct((NP, D), jnp.float32),
  )(agg, bp)


def _make_sc_edge(single_head):
  mesh = plsc.VectorSubcoreMesh(core_axis_name="c", subcore_axis_name="s")
  buf_types = [
      pltpu.VMEM((K,), jnp.int32),
      pltpu.VMEM((K,), jnp.int32),
      pltpu.VMEM((K,), jnp.int32),
      pltpu.VMEM((K,), jnp.int32),
      pltpu.VMEM((K, 16), jnp.float32),
      pltpu.VMEM((K, 16), jnp.float32),
      pltpu.VMEM((K, 128), jnp.float32),
      pltpu.VMEM((K,), jnp.int32),
      pltpu.VMEM((K, 144), jnp.float32),
      pltpu.SemaphoreType.DMA,
      pltpu.SemaphoreType.DMA,
      pltpu.SemaphoreType.DMA,
      pltpu.SemaphoreType.DMA,
  ]

  @functools.partial(
      pl.kernel,
      mesh=mesh,
      out_type=jax.ShapeDtypeStruct((2 * NP, 144), jnp.float32),
      scratch_types=buf_types + buf_types + [
          pltpu.VMEM_SHARED((NPA, 144), jnp.float32),
      ],
      compiler_params=pltpu.CompilerParams(use_tc_tiling_on_sc=False),
  )
  def k(src_h, dst_h, as_h, ad_h, xl_h, agg_h, *refs):
    nb = len(refs) // 2
    bufA = refs[0:nb]
    bufB = refs[nb:2 * nb]
    ov = bufA[8]
    acc = refs[2 * nb]
    c = lax.axis_index("c")
    s = lax.axis_index("s")
    zero16 = jnp.zeros((16,), jnp.float32)

    def zrow(i, carry):
      for j in range(9):
        ov[i, pl.ds(j * 16, 16)] = zero16
      return carry
    lax.fori_loop(0, K, zrow, 0)
    r0 = s * ROWS_PER_TILE
    for t in range(ROWS_PER_TILE // K):
      pltpu.sync_copy(ov, acc.at[pl.ds(r0 + t * K, K)])
    tail = ROWS_PER_TILE % K
    if tail:
      pltpu.sync_copy(ov.at[pl.ds(0, tail)],
                      acc.at[pl.ds(r0 + (ROWS_PER_TILE // K) * K, tail)])
    plsc.subcore_barrier()

    lane = lax.iota(jnp.int32, 16)
    coff = c * NP
    JH = 1 if single_head else 4
    VPERH = 8 if single_head else 2
    lane_mask = lane < JH

    ebase = s * (CH * K)

    def prefetch(g, buf):
      si, di, si2, di2, avb, bvb, xvb, dis, ovb, sa, sb, sx, ss = buf
      b0 = ebase + g * K
      pltpu.sync_copy(src_h.at[pl.ds(b0, K)], si)
      pltpu.sync_copy(dst_h.at[pl.ds(b0, K)], di)
      for j in range(K // 16):
        si2[pl.ds(j * 16, 16)] = si[pl.ds(j * 16, 16)] + coff
        di2[pl.ds(j * 16, 16)] = di[pl.ds(j * 16, 16)] + coff
      pltpu.async_copy(as_h.at[si2], avb, sa)
      pltpu.async_copy(ad_h.at[di2], bvb, sb)
      pltpu.async_copy(xl_h.at[si2], xvb, sx)

    def consume(g, t, buf):
      si, di, si2, di2, avb, bvb, xvb, dis, ovb, sa, sb, sx, ss = buf
      pltpu.make_async_copy(as_h.at[si2], avb, sa).wait()
      pltpu.make_async_copy(ad_h.at[di2], bvb, sb).wait()
      pltpu.make_async_copy(xl_h.at[si2], xvb, sx).wait()

      @pl.when(t > 0)
      def _():
        pltpu.make_async_copy(ovb, acc.at[dis], ss).wait()

      @plsc.parallel_loop(0, K, unroll=4)
      def edge(kk):
        ev = avb[kk, :] + bvb[kk, :]
        w16 = jnp.exp(jnp.maximum(ev, 0.2 * ev))
        ovb[kk, pl.ds(128, 16)] = jnp.where(lane_mask, w16, zero16)
        for jh in range(JH):
          w_v = jnp.full((16,), w16[jh], jnp.float32)
          for j in range(VPERH):
            col = (jh * VPERH + j) * 16
            ovb[kk, pl.ds(col, 16)] = xvb[kk, pl.ds(col, 16)] * w_v
      for j in range(K // 16):
        dis[pl.ds(j * 16, 16)] = di[pl.ds(j * 16, 16)]
      pltpu.async_copy(ovb, acc.at[dis], ss, add=True)

    prefetch(0, bufA)
    prefetch(1, bufB)

    def pair(t, carry):
      g = 2 * t
      consume(g, t, bufA)

      @pl.when(g + 2 < CH)
      def _():
        prefetch(g + 2, bufA)
      consume(g + 1, t, bufB)

      @pl.when(g + 3 < CH)
      def _():
        prefetch(g + 3, bufB)
      return carry
    lax.fori_loop(0, CH // 2, pair, 0)

    siA, diA, si2A, di2A, avA, bvA, xvA, disA, ovA, saA, sbA, sxA, ssA = bufA
    siB, diB, si2B, di2B, avB, bvB, xvB, disB, ovB, saB, sbB, sxB, ssB = bufB
    pltpu.make_async_copy(ovA, acc.at[disA], ssA).wait()
    pltpu.make_async_copy(ovB, acc.at[disB], ssB).wait()

    plsc.subcore_barrier()
    pltpu.sync_copy(acc.at[pl.ds(r0, ROWS_PER_TILE)],
                    agg_h.at[pl.ds(coff + r0, ROWS_PER_TILE)])

  return k


_sc_edge_multi = _make_sc_edge(False)
_sc_edge_single = _make_sc_edge(True)


def _alpha_mat(a, heads):
  a_flat = a.reshape(-1)
  lvec = jnp.arange(16, dtype=jnp.int32)
  if heads == 1:
    slab = a_flat[:, None] * (lvec[None, :] == 0).astype(jnp.float32)
    return jnp.stack([slab, slab])
  hvec = jnp.arange(D, dtype=jnp.int32) // C
  slabs = []
  for c in range(2):
    mask = (lvec[None, :] == hvec[:, None] - 4 * c).astype(jnp.float32)
    slabs.append(a_flat[:, None] * mask)
  return jnp.stack(slabs)


def kernel(x, edge_index, W1, a_src1, a_dst1, b1, gamma1, beta1,
           W2, a_src2, a_dst2, b2, gamma2, beta2,
           W3, a_src3, a_dst3, b3):
  f32 = jnp.float32
  xp = jnp.pad(x, ((0, NP - N), (0, 0)))
  loop = jnp.arange(N, dtype=jnp.int32)
  pad = jnp.full((ET - edge_index.shape[1] - N,), N, dtype=jnp.int32)
  src = jnp.concatenate([edge_index[0].astype(jnp.int32), loop, pad])
  dst = jnp.concatenate([edge_index[1].astype(jnp.int32), loop, pad])

  inv = 1.0 / jnp.sqrt(jnp.asarray(1.0 + 1e-5, f32))
  g1p = (gamma1 * inv)[None, :]
  b1p = (b1 * gamma1 * inv + beta1)[None, :]
  g2p = (gamma2 * inv)[None, :]
  b2p = (b2 * gamma2 * inv + beta2)[None, :]
  b3p = b3[None, :]

  As1 = _alpha_mat(a_src1, H)
  Ad1 = _alpha_mat(a_dst1, H)
  As2 = _alpha_mat(a_src2, H)
  Ad2 = _alpha_mat(a_dst2, H)
  As3 = _alpha_mat(a_src3, 1)
  Ad3 = _alpha_mat(a_dst3, 1)

  xl1, as1, ad1 = _tc_entry(xp, W1, As1, Ad1)
  agg1 = _sc_edge_multi(src, dst, as1.reshape(2 * NP, 16),
                        ad1.reshape(2 * NP, 16), xl1.reshape(2 * NP, 128))
  xl2, as2, ad2 = _tc_mid(agg1.reshape(2, NP, 144), g1p, b1p, W2, As2, Ad2)
  agg2 = _sc_edge_multi(src, dst, as2.reshape(2 * NP, 16),
                        ad2.reshape(2 * NP, 16), xl2.reshape(2 * NP, 128))
  xl3, as3, ad3 = _tc_mid(agg2.reshape(2, NP, 144), g2p, b2p, W3, As3, Ad3)
  agg3 = _sc_edge_single(src, dst, as3.reshape(2 * NP, 16),
                         ad3.reshape(2 * NP, 16), xl3.reshape(2 * NP, 128))
  out = _tc_final(agg3.reshape(2, NP, 144), b3p)
  return out[:N]
